# Initial kernel scaffold; baseline (speedup 1.0000x reference)
import jax
import jax.numpy as jnp
from jax.experimental import pallas as pl


def kernel(tokens, oov_features, fixed_weights):
    raise NotImplementedError("write your pallas kernel here")


# speedup vs baseline: 5.3088x; 5.3088x over previous
import functools

import jax
import jax.numpy as jnp
from jax import lax
from jax.experimental import pallas as pl
from jax.experimental.pallas import tpu as pltpu
from jax.experimental.pallas import tpu_sc as plsc

PADDING_IDX = 0
LANES = 16


def _make_sc_gather(vocab, n_oov_total, d, rpad, rpw, nc, ns):
    mesh = plsc.VectorSubcoreMesh(core_axis_name="c", subcore_axis_name="s")

    @functools.partial(
        pl.kernel,
        mesh=mesh,
        out_type=jax.ShapeDtypeStruct((rpad, d), jnp.float32),
        compiler_params=pltpu.CompilerParams(use_tc_tiling_on_sc=False),
        scratch_types=[
            pltpu.VMEM((rpw,), jnp.int32),
            pltpu.VMEM((rpw,), jnp.int32),
            pltpu.VMEM((rpw,), jnp.int32),
            pltpu.VMEM((rpw,), jnp.int32),
            pltpu.VMEM((rpw,), jnp.int32),
            pltpu.VMEM((rpw, d), jnp.float32),
            pltpu.VMEM((rpw, d), jnp.float32),
            pltpu.VMEM((rpw, d), jnp.float32),
            pltpu.SemaphoreType.DMA,
            pltpu.SemaphoreType.DMA,
            pltpu.SemaphoreType.DMA,
        ],
    )
    def gather_kernel(tok_hbm, bb_hbm, fw_hbm, oov_hbm, mtab_hbm, out_hbm,
                      tok_v, bb_v, idxf_v, idxo_v, sel_v, rf_v, ro_v, rm_v,
                      s0, s1, s2):
        wid = lax.axis_index("s") * nc + lax.axis_index("c")
        base = wid * rpw
        pltpu.sync_copy(tok_hbm.at[pl.ds(base, rpw)], tok_v)
        pltpu.sync_copy(bb_hbm.at[pl.ds(base, rpw)], bb_v)
        for c in range(rpw // LANES):
            sl = pl.ds(c * LANES, LANES)
            t = tok_v[sl]
            bb = bb_v[sl]
            m = t >= vocab
            idxf_v[sl] = jnp.where(m, 0, t)
            idxo_v[sl] = jnp.where(m, t - vocab + bb, 0)
            sel_v[sl] = jnp.where(m, 1, 0)
        cp_f = pltpu.async_copy(fw_hbm.at[idxf_v], rf_v, s0)
        cp_o = pltpu.async_copy(oov_hbm.at[idxo_v], ro_v, s1)
        cp_m = pltpu.async_copy(mtab_hbm.at[sel_v], rm_v, s2)
        cp_f.wait()
        cp_o.wait()
        cp_m.wait()
        for r in range(rpw):
            for c in range(d // LANES):
                sl = pl.ds(c * LANES, LANES)
                f = rf_v[r, sl]
                o = ro_v[r, sl]
                mm = rm_v[r, sl]
                rf_v[r, sl] = jnp.where(mm > 0.5, o, f)
        pltpu.sync_copy(rf_v, out_hbm.at[pl.ds(base, rpw)])

    return gather_kernel


def kernel(tokens, oov_features, fixed_weights):
    bs, seq = tokens.shape
    n_oov = oov_features.shape[1]
    vocab, d = fixed_weights.shape
    rows = bs * seq

    info = plsc.get_sparse_core_info()
    nc, ns = info.num_cores, info.num_subcores
    nw = nc * ns
    rpw = -(-rows // nw)
    rpw = -(-rpw // LANES) * LANES
    rpad = nw * rpw

    tok_flat = jnp.pad(tokens.reshape(-1), (0, rpad - rows))
    bb = (jnp.arange(rpad, dtype=jnp.int32) // seq).clip(0, bs - 1) * n_oov
    oov_flat = oov_features.reshape(bs * n_oov, d)
    mtab = jnp.stack([jnp.zeros((d,), jnp.float32), jnp.ones((d,), jnp.float32)])

    gather = _make_sc_gather(vocab, bs * n_oov, d, rpad, rpw, nc, ns)
    out = gather(tok_flat, bb, fixed_weights, oov_flat, mtab)

    features = out[:rows].reshape(bs, seq, d)
    padding_mask = (tokens == PADDING_IDX)[:, None, None, :]
    sequential_mask = jnp.triu(jnp.ones((seq, seq), dtype=bool), k=1)
    return (features, padding_mask, sequential_mask)


# speedup vs baseline: 8.9072x; 1.6778x over previous
import functools

import jax
import jax.numpy as jnp
from jax import lax
from jax.experimental import pallas as pl
from jax.experimental.pallas import tpu as pltpu
from jax.experimental.pallas import tpu_sc as plsc

PADDING_IDX = 0
LANES = 16


def _make_sc_gather(vocab, d, rpad, rpw, nc, ns):
    mesh = plsc.VectorSubcoreMesh(core_axis_name="c", subcore_axis_name="s")

    @functools.partial(
        pl.kernel,
        mesh=mesh,
        out_type=jax.ShapeDtypeStruct((rpad, d), jnp.float32),
        scratch_types=[
            pltpu.VMEM((rpw,), jnp.int32),
            pltpu.VMEM((rpw,), jnp.int32),
            pltpu.VMEM((rpw,), jnp.int32),
            pltpu.SMEM((rpw,), jnp.int32),
            pltpu.VMEM((rpw, d), jnp.float32),
            pltpu.SemaphoreType.DMA,
        ],
    )
    def gather_kernel(tok_hbm, bb_hbm, fw_hbm, oov_hbm, out_hbm,
                      tok_v, bb_v, comb_v, comb_s, out_v, s0):
        wid = lax.axis_index("s") * nc + lax.axis_index("c")
        base = wid * rpw
        pltpu.sync_copy(tok_hbm.at[pl.ds(base, rpw)], tok_v)
        pltpu.sync_copy(bb_hbm.at[pl.ds(base, rpw)], bb_v)
        for c in range(rpw // LANES):
            sl = pl.ds(c * LANES, LANES)
            t = tok_v[sl]
            comb_v[sl] = jnp.where(t >= vocab, t + bb_v[sl], t)
        for g in range(rpw // LANES):
            v16 = comb_v[pl.ds(g * LANES, LANES)]
            for r in range(LANES):
                i = g * LANES + r
                t = v16[r]

                @pl.when(t < vocab)
                def _():
                    pltpu.async_copy(fw_hbm.at[t], out_v.at[i], s0)

                @pl.when(t >= vocab)
                def _():
                    pltpu.async_copy(oov_hbm.at[t - vocab], out_v.at[i], s0)

        pltpu.make_async_copy(out_hbm.at[pl.ds(base, rpw)], out_v, s0).wait()
        pltpu.sync_copy(out_v, out_hbm.at[pl.ds(base, rpw)])

    return gather_kernel


def kernel(tokens, oov_features, fixed_weights):
    bs, seq = tokens.shape
    n_oov = oov_features.shape[1]
    vocab, d = fixed_weights.shape
    rows = bs * seq

    info = plsc.get_sparse_core_info()
    nc, ns = info.num_cores, info.num_subcores
    nw = nc * ns
    rpw = -(-rows // nw)
    rpw = -(-rpw // LANES) * LANES
    rpad = nw * rpw

    tok_flat = jnp.pad(tokens.reshape(-1), (0, rpad - rows))
    bb = (jnp.arange(rpad, dtype=jnp.int32) // seq).clip(0, bs - 1) * n_oov
    oov_flat = oov_features.reshape(bs * n_oov, d)

    gather = _make_sc_gather(vocab, d, rpad, rpw, nc, ns)
    out = gather(tok_flat, bb, fixed_weights, oov_flat)

    features = out[:rows].reshape(bs, seq, d)
    padding_mask = (tokens == PADDING_IDX)[:, None, None, :]
    sequential_mask = jnp.triu(jnp.ones((seq, seq), dtype=bool), k=1)
    return (features, padding_mask, sequential_mask)


# speedup vs baseline: 10.4744x; 1.1760x over previous
import functools

import jax
import jax.numpy as jnp
from jax import lax
from jax.experimental import pallas as pl
from jax.experimental.pallas import tpu as pltpu
from jax.experimental.pallas import tpu_sc as plsc

PADDING_IDX = 0
LANES = 16


def _make_sc_gather(vocab, d, rpad, rpw, nc, ns):
    mesh = plsc.VectorSubcoreMesh(core_axis_name="c", subcore_axis_name="s", num_cores=1)

    @functools.partial(
        pl.kernel,
        mesh=mesh,
        out_type=jax.ShapeDtypeStruct((rpad, d), jnp.float32),
        scratch_types=[
            pltpu.VMEM((rpw,), jnp.int32),
            pltpu.VMEM((rpw,), jnp.int32),
            pltpu.VMEM((rpw,), jnp.int32),
            pltpu.SMEM((rpw,), jnp.int32),
            pltpu.VMEM((rpw, d), jnp.float32),
            pltpu.SemaphoreType.DMA,
        ],
    )
    def gather_kernel(tok_hbm, bb_hbm, fw_hbm, oov_hbm, out_hbm,
                      tok_v, bb_v, comb_v, comb_s, out_v, s0):
        wid = lax.axis_index("s") * nc + lax.axis_index("c")
        base = wid * rpw
        pltpu.sync_copy(tok_hbm.at[pl.ds(base, rpw)], tok_v)
        pltpu.sync_copy(bb_hbm.at[pl.ds(base, rpw)], bb_v)
        for c in range(rpw // LANES):
            sl = pl.ds(c * LANES, LANES)
            t = tok_v[sl]
            comb_v[sl] = jnp.where(t >= vocab, t + bb_v[sl], t)
        for g in range(rpw // LANES):
            v16 = comb_v[pl.ds(g * LANES, LANES)]
            for r in range(LANES):
                i = g * LANES + r
                t = v16[r]

                @pl.when(t < vocab)
                def _():
                    pltpu.async_copy(fw_hbm.at[t], out_v.at[i], s0)

                @pl.when(t >= vocab)
                def _():
                    pltpu.async_copy(oov_hbm.at[t - vocab], out_v.at[i], s0)

        pltpu.make_async_copy(out_hbm.at[pl.ds(base, rpw)], out_v, s0).wait()
        pltpu.sync_copy(out_v, out_hbm.at[pl.ds(base, rpw)])

    return gather_kernel


def kernel(tokens, oov_features, fixed_weights):
    bs, seq = tokens.shape
    n_oov = oov_features.shape[1]
    vocab, d = fixed_weights.shape
    rows = bs * seq

    info = plsc.get_sparse_core_info()
    nc, ns = info.num_cores, info.num_subcores
    nw = nc * ns
    rpw = -(-rows // nw)
    rpw = -(-rpw // LANES) * LANES
    rpad = nw * rpw

    tok_flat = jnp.pad(tokens.reshape(-1), (0, rpad - rows))
    bb = (jnp.arange(rpad, dtype=jnp.int32) // seq).clip(0, bs - 1) * n_oov
    oov_flat = oov_features.reshape(bs * n_oov, d)

    gather = _make_sc_gather(vocab, d, rpad, rpw, nc, ns)
    out = gather(tok_flat, bb, fixed_weights, oov_flat)

    features = out[:rows].reshape(bs, seq, d)
    padding_mask = (tokens == PADDING_IDX)[:, None, None, :]
    sequential_mask = jnp.triu(jnp.ones((seq, seq), dtype=bool), k=1)
    return (features, padding_mask, sequential_mask)
